# Initial kernel scaffold; baseline (speedup 1.0000x reference)
import jax
import jax.numpy as jnp
from jax.experimental import pallas as pl


def kernel(x, emb_table, lin_W, att_i, att_j, att_em_i, att_em_j, gl_bias, bn_gamma, bn_beta, out_W, out_b):
    raise NotImplementedError("write your pallas kernel here")


# speedup vs baseline: 1.0002x; 1.0002x over previous
import functools

import jax
import jax.numpy as jnp
from jax.experimental import pallas as pl
from jax.experimental.pallas import tpu as pltpu

K = 16


def _lin_body(x_ref, w_ref, o_ref):
    o_ref[...] = jnp.dot(x_ref[...], w_ref[...],
                         preferred_element_type=jnp.float32)


def _lin(xb, lin_WT):
    BN, L = xb.shape
    D = lin_WT.shape[1]
    blk = 2000
    return pl.pallas_call(
        _lin_body,
        grid=(BN // blk,),
        in_specs=[
            pl.BlockSpec((blk, L), lambda i: (i, 0)),
            pl.BlockSpec((L, D), lambda i: (0, 0)),
        ],
        out_specs=pl.BlockSpec((blk, D), lambda i: (i, 0)),
        out_shape=jax.ShapeDtypeStruct((BN, D), jnp.float32),
    )(xb, lin_WT)


def kernel(x, emb_table, lin_W, att_i, att_j, att_em_i, att_em_j, gl_bias,
           bn_gamma, bn_beta, out_W, out_b):
    B, N, L = x.shape
    D = emb_table.shape[1]
    BN = B * N

    nrm = jnp.linalg.norm(emb_table, axis=-1, keepdims=True)
    sim = (emb_table @ emb_table.T) / (nrm @ nrm.T)
    _, idx = jax.lax.top_k(sim, K)
    src = jnp.repeat(jnp.arange(N), K)
    dst = idx.reshape(-1)
    e = jnp.stack([src, dst], axis=0)
    e = jnp.concatenate([e + i * N for i in range(B)], axis=1)
    mask = e[0] != e[1]
    e = e.at[1].set(jnp.where(mask, e[1], BN))
    loops = jnp.arange(BN)
    e = jnp.concatenate([e, jnp.stack([loops, loops])], axis=1)
    src, tgt = e[0], e[1]

    xb = x.reshape(BN, L)
    emb_b = jnp.tile(emb_table, (B, 1))
    g = _lin(xb, lin_W.T)

    x_i = g[tgt]
    x_j = g[src]
    emb_i = emb_b[tgt]
    emb_j = emb_b[src]
    key_i = jnp.concatenate([x_i, emb_i], axis=-1)
    key_j = jnp.concatenate([x_j, emb_j], axis=-1)
    a_i = jnp.concatenate([att_i, att_em_i])
    a_j = jnp.concatenate([att_j, att_em_j])
    alpha = (key_i * a_i).sum(-1) + (key_j * a_j).sum(-1)
    alpha = jax.nn.leaky_relu(alpha, 0.2)
    amax = jax.ops.segment_max(alpha, tgt, num_segments=BN + 1)
    alpha = jnp.exp(alpha - jax.lax.stop_gradient(amax)[tgt])
    denom = jax.ops.segment_sum(alpha, tgt, num_segments=BN + 1)
    alpha = alpha / (denom[tgt] + 1e-16)
    msg = x_j * alpha[:, None]
    z = jax.ops.segment_sum(msg, tgt, num_segments=BN + 1)[:BN] + gl_bias

    s = z * emb_b
    s = s / jnp.sqrt(1.0 + 1e-5) * bn_gamma + bn_beta
    s = jax.nn.relu(s)
    out = s @ out_W.T + out_b
    return out.reshape(B, N)


# speedup vs baseline: 1.7142x; 1.7139x over previous
import functools

import jax
import jax.numpy as jnp
from jax import lax
from jax.experimental import pallas as pl
from jax.experimental.pallas import tpu as pltpu
from jax.experimental.pallas import tpu_sc as plsc

K = 16
_BNK = float(1.0 / (1.0 + 1e-5) ** 0.5)

_NP = 10240
_RPT = _NP // 16
_GRP = 8
_NGRP = _RPT // _GRP


def _edge_body(idx_hbm, at_hbm, bs_hbm, g_hbm, z_hbm, num_hbm, den_hbm,
               idx_v, at_v, bs_v, gbuf_v, msg_v, tgt_v, den_v, num_sh):
    c = lax.axis_index("c")
    s = lax.axis_index("s")
    base = s * _RPT
    cb = c * _NP

    pltpu.sync_copy(idx_hbm.at[pl.ds(base * K, _RPT * K)], idx_v)
    pltpu.sync_copy(at_hbm.at[pl.ds(cb, _NP)], at_v)
    pltpu.sync_copy(bs_hbm.at[pl.ds(cb + base, _RPT)], bs_v.at[pl.ds(0, _RPT)])
    pltpu.sync_copy(z_hbm, num_sh.at[pl.ds(base, _RPT)])
    zv = jnp.zeros((16,), jnp.float32)

    def zrow(i, _):
        den_v[pl.ds(i * 16, 16)] = zv
        return None

    lax.fori_loop(0, _NP // 16, zrow, None)
    plsc.subcore_barrier()

    def group(gg, _):
        gb = base + gg * _GRP
        pltpu.sync_copy(g_hbm.at[pl.ds((cb + gb) * 128, _GRP * 128)], gbuf_v)
        b16 = bs_v[pl.ds(gg * _GRP, 16)]
        for jj in range(_GRP):
            rl = gg * _GRP + jj
            rowid = base + rl
            tgt16 = idx_v[pl.ds(rl * K, 16)]
            a16 = plsc.load_gather(at_v, [tgt16])
            al = a16 + jnp.broadcast_to(b16[jj], (16,))
            al = jnp.where(al >= 0, al, 0.2 * al)
            ok = jnp.full((16,), rowid < 10000) & (tgt16 != jnp.full((16,), rowid))
            w = jnp.where(ok, jnp.exp(al), 0.0)
            plsc.addupdate_scatter(den_v, [tgt16], w)
            tgt_v[pl.ds(jj * 16, 16)] = tgt16
            gs = [gbuf_v[pl.ds(jj * 128 + seg * 16, 16)] for seg in range(8)]
            for kk in range(16):
                wk = jnp.broadcast_to(w[kk], (16,))
                row = jj * 16 + kk
                for seg in range(8):
                    msg_v[row, pl.ds(seg * 16, 16)] = gs[seg] * wk
        pltpu.sync_copy(msg_v, num_sh.at[tgt_v], add=True)

    lax.fori_loop(0, _NGRP, group, None)
    pltpu.sync_copy(den_v, den_hbm.at[c, s, :])
    plsc.subcore_barrier()
    pltpu.sync_copy(num_sh.at[pl.ds(base, _RPT)],
                    num_hbm.at[c, pl.ds(base, _RPT), :])


def _edge_sc(idx_flat, a_t2, b_s2, g2flat, zeros_init):
    mesh = plsc.VectorSubcoreMesh(core_axis_name="c", subcore_axis_name="s")
    f = functools.partial(
        pl.kernel, mesh=mesh,
        out_type=[
            jax.ShapeDtypeStruct((2, _NP, 128), jnp.float32),
            jax.ShapeDtypeStruct((2, 16, _NP), jnp.float32),
        ],
        compiler_params=pltpu.CompilerParams(needs_layout_passes=False),
        scratch_types=[
            pltpu.VMEM((_RPT * K,), jnp.int32),
            pltpu.VMEM((_NP,), jnp.float32),
            pltpu.VMEM((_RPT + 16,), jnp.float32),
            pltpu.VMEM((_GRP * 128,), jnp.float32),
            pltpu.VMEM((_GRP * 16, 128), jnp.float32),
            pltpu.VMEM((_GRP * 16,), jnp.int32),
            pltpu.VMEM((_NP,), jnp.float32),
            pltpu.VMEM_SHARED((_NP, 128), jnp.float32),
        ],
    )(_edge_body)
    return f(idx_flat, a_t2, b_s2, g2flat, zeros_init)


def _linscal_body(x_ref, emb_ref, w_ref, att_ref, g_ref, at_ref, bs_ref, ws_ref):
    g = jnp.dot(x_ref[...], w_ref[...], preferred_element_type=jnp.float32)
    g_ref[...] = g
    emb = emb_ref[...]
    att = att_ref[...]
    a_t = g @ att[0, :] + emb @ att[2, :]
    b_s = g @ att[1, :] + emb @ att[3, :]
    at_ref[...] = a_t[None, None, :]
    bs_ref[...] = b_s[None, None, :]
    a = a_t + b_s
    ws_ref[...] = jnp.exp(jnp.where(a >= 0, a, 0.2 * a))[None, None, :]


def _linscal(xb, emb_b, lin_WT, att4):
    BN, L = xb.shape
    D = lin_WT.shape[1]
    blk = 2000
    grid = (BN // blk,)
    g, a_t, b_s, wself = pl.pallas_call(
        _linscal_body,
        grid=grid,
        in_specs=[
            pl.BlockSpec((blk, L), lambda i: (i, 0)),
            pl.BlockSpec((blk, D), lambda i: (i, 0)),
            pl.BlockSpec((L, D), lambda i: (0, 0)),
            pl.BlockSpec((4, D), lambda i: (0, 0)),
        ],
        out_specs=[
            pl.BlockSpec((blk, D), lambda i: (i, 0)),
            pl.BlockSpec((1, 1, blk), lambda i: (i, 0, 0)),
            pl.BlockSpec((1, 1, blk), lambda i: (i, 0, 0)),
            pl.BlockSpec((1, 1, blk), lambda i: (i, 0, 0)),
        ],
        out_shape=[
            jax.ShapeDtypeStruct((BN, D), jnp.float32),
            jax.ShapeDtypeStruct((BN // blk, 1, blk), jnp.float32),
            jax.ShapeDtypeStruct((BN // blk, 1, blk), jnp.float32),
            jax.ShapeDtypeStruct((BN // blk, 1, blk), jnp.float32),
        ],
    )(xb, emb_b, lin_WT, att4)
    return g, a_t.reshape(BN), b_s.reshape(BN), wself.reshape(BN)


def _head_body(num_ref, den_ref, ws_ref, g_ref, emb_ref, gb_ref, bn_ref,
               ow_ref, ob_ref, o_ref):
    ws = ws_ref[0, 0, :]
    den = jnp.sum(den_ref[0], axis=1) + ws + 1e-16
    z = (num_ref[0] + ws[:, None] * g_ref[...]) / den[:, None]
    z = z + gb_ref[...]
    s = z * emb_ref[...]
    s = s * _BNK * bn_ref[0:1, :] + bn_ref[1:2, :]
    s = jnp.maximum(s, 0.0)
    t = jnp.sum(s * ow_ref[...], axis=1)
    o_ref[...] = (t + ob_ref[0, 0])[None, None, :]


def _head(num, den16, wself, g, emb_b, gl_bias, bn2, out_W, out_b):
    D = g.shape[1]
    BN = g.shape[0]
    blk = 2000
    nb = _NP // blk
    out = pl.pallas_call(
        _head_body,
        grid=(BN // blk,),
        in_specs=[
            pl.BlockSpec((1, blk, D), lambda i: (i // nb, i % nb, 0)),
            pl.BlockSpec((1, blk, 16), lambda i: (i // nb, i % nb, 0)),
            pl.BlockSpec((1, 1, blk), lambda i: (i, 0, 0)),
            pl.BlockSpec((blk, D), lambda i: (i, 0)),
            pl.BlockSpec((blk, D), lambda i: (i, 0)),
            pl.BlockSpec((1, D), lambda i: (0, 0)),
            pl.BlockSpec((2, D), lambda i: (0, 0)),
            pl.BlockSpec((1, D), lambda i: (0, 0)),
            pl.BlockSpec((1, 1), lambda i: (0, 0)),
        ],
        out_specs=pl.BlockSpec((1, 1, blk), lambda i: (i, 0, 0)),
        out_shape=jax.ShapeDtypeStruct((BN // blk, 1, blk), jnp.float32),
    )(num, den16, wself.reshape(BN // blk, 1, blk),
      g, emb_b, gl_bias.reshape(1, D), bn2, out_W, out_b.reshape(1, 1))
    return out.reshape(BN)


def kernel(x, emb_table, lin_W, att_i, att_j, att_em_i, att_em_j, gl_bias,
           bn_gamma, bn_beta, out_W, out_b):
    B, N, L = x.shape
    D = emb_table.shape[1]
    BN = B * N

    nrm = jnp.linalg.norm(emb_table, axis=-1, keepdims=True)
    sim = (emb_table @ emb_table.T) / (nrm @ nrm.T)
    _, idx = jax.lax.top_k(sim, K)

    xb = x.reshape(BN, L)
    emb_b = jnp.tile(emb_table, (B, 1))
    att4 = jnp.stack([att_i, att_j, att_em_i, att_em_j], axis=0)
    g, a_t, b_s, wself = _linscal(xb, emb_b, lin_W.T, att4)

    idx_p = jnp.zeros((_NP, K), jnp.int32).at[:N].set(idx.astype(jnp.int32))
    a_t2 = jnp.zeros((B, _NP), jnp.float32).at[:, :N].set(a_t.reshape(B, N))
    b_s2 = jnp.zeros((B, _NP), jnp.float32).at[:, :N].set(b_s.reshape(B, N))
    g2 = jnp.zeros((B, _NP, D), jnp.float32).at[:, :N].set(g.reshape(B, N, D))
    zeros_init = jnp.zeros((_RPT, D), jnp.float32)
    num_out, den_out = _edge_sc(idx_p.reshape(-1), a_t2.reshape(-1),
                                b_s2.reshape(-1), g2.reshape(-1), zeros_init)
    dent = den_out.transpose(0, 2, 1)

    bn2 = jnp.stack([bn_gamma, bn_beta], axis=0)
    out = _head(num_out, dent, wself, g, emb_b, gl_bias, bn2, out_W, out_b)
    return out.reshape(B, N)


# speedup vs baseline: 2.3559x; 1.3743x over previous
import functools

import jax
import jax.numpy as jnp
from jax import lax
from jax.experimental import pallas as pl
from jax.experimental.pallas import tpu as pltpu
from jax.experimental.pallas import tpu_sc as plsc

K = 16
_BNK = float(1.0 / (1.0 + 1e-5) ** 0.5)

_NP = 10240
_RPT = _NP // 16
_GRP = 8
_NGRP = _RPT // _GRP


def _edge_body(idx_hbm, at_hbm, bs_hbm, g_hbm, z_hbm, num_hbm, den_hbm,
               idx_v, at_v, bs_v, gbuf_v, msg_v, tgt_v, den_v, num_sh):
    c = lax.axis_index("c")
    s = lax.axis_index("s")
    base = s * _RPT
    cb = c * _NP

    pltpu.sync_copy(idx_hbm.at[pl.ds(base * K, _RPT * K)], idx_v)
    pltpu.sync_copy(at_hbm.at[pl.ds(cb, _NP)], at_v)
    pltpu.sync_copy(bs_hbm.at[pl.ds(cb + base, _RPT)], bs_v.at[pl.ds(0, _RPT)])
    pltpu.sync_copy(z_hbm, num_sh.at[pl.ds(base, _RPT)])
    zv = jnp.zeros((16,), jnp.float32)

    def zrow(i, _):
        den_v[pl.ds(i * 16, 16)] = zv
        return None

    lax.fori_loop(0, _NP // 16, zrow, None)
    plsc.subcore_barrier()

    def group(gg, _):
        gb = base + gg * _GRP
        pltpu.sync_copy(g_hbm.at[pl.ds((cb + gb) * 128, _GRP * 128)], gbuf_v)
        b16 = bs_v[pl.ds(gg * _GRP, 16)]
        for jj in range(_GRP):
            rl = gg * _GRP + jj
            rowid = base + rl
            okr = jnp.full((16,), rowid < 10000)
            tgt16 = jnp.where(okr, idx_v[pl.ds(rl * K, 16)], 0)
            a16 = plsc.load_gather(at_v, [tgt16])
            al = a16 + jnp.broadcast_to(b16[jj], (16,))
            al = jnp.where(al >= 0, al, 0.2 * al)
            ok = okr & (tgt16 != jnp.full((16,), rowid))
            w = jnp.where(ok, jnp.exp(al), 0.0)
            plsc.addupdate_scatter(den_v, [tgt16], w)
            tgt_v[pl.ds(jj * 16, 16)] = tgt16
            gs = [gbuf_v[pl.ds(jj * 128 + seg * 16, 16)] for seg in range(8)]
            for kk in range(16):
                wk = jnp.broadcast_to(w[kk], (16,))
                row = jj * 16 + kk
                for seg in range(8):
                    msg_v[row, pl.ds(seg * 16, 16)] = gs[seg] * wk
        pltpu.sync_copy(msg_v, num_sh.at[tgt_v], add=True)

    lax.fori_loop(0, _NGRP, group, None)
    pltpu.sync_copy(den_v, den_hbm.at[c, s, :])
    plsc.subcore_barrier()
    pltpu.sync_copy(num_sh.at[pl.ds(base, _RPT)],
                    num_hbm.at[c, pl.ds(base, _RPT), :])


def _edge_sc(idx_flat, a_t2, b_s2, g2flat, zeros_init):
    mesh = plsc.VectorSubcoreMesh(core_axis_name="c", subcore_axis_name="s")
    f = functools.partial(
        pl.kernel, mesh=mesh,
        out_type=[
            jax.ShapeDtypeStruct((2, _NP, 128), jnp.float32),
            jax.ShapeDtypeStruct((2, 16, _NP), jnp.float32),
        ],
        compiler_params=pltpu.CompilerParams(needs_layout_passes=False),
        scratch_types=[
            pltpu.VMEM((_RPT * K,), jnp.int32),
            pltpu.VMEM((_NP,), jnp.float32),
            pltpu.VMEM((_RPT + 16,), jnp.float32),
            pltpu.VMEM((_GRP * 128,), jnp.float32),
            pltpu.VMEM((_GRP * 16, 128), jnp.float32),
            pltpu.VMEM((_GRP * 16,), jnp.int32),
            pltpu.VMEM((_NP,), jnp.float32),
            pltpu.VMEM_SHARED((_NP, 128), jnp.float32),
        ],
    )(_edge_body)
    return f(idx_flat, a_t2, b_s2, g2flat, zeros_init)


def _norm_body(e_ref, o_ref):
    ss = jnp.sum(e_ref[...] * e_ref[...], axis=1)
    o_ref[...] = jnp.where(ss > 0, lax.rsqrt(ss), 0.0)[None, None, :]


def _norms(embp):
    NPAD, D = embp.shape
    blk = 2048
    out = pl.pallas_call(
        _norm_body,
        grid=(NPAD // blk,),
        in_specs=[pl.BlockSpec((blk, D), lambda i: (i, 0))],
        out_specs=pl.BlockSpec((1, 1, blk), lambda i: (i, 0, 0)),
        out_shape=jax.ShapeDtypeStruct((NPAD // blk, 1, blk), jnp.float32),
    )(embp)
    return out.reshape(NPAD)


_RB, _CB = 256, 512


def _sim_body(er_ref, ec_ref, ir_ref, ic_ref, sim_ref, cm_ref):
    j = pl.program_id(1)
    d = lax.dot_general(er_ref[...], ec_ref[...], (((1,), (1,)), ((), ())),
                        preferred_element_type=jnp.float32)
    s = d * ir_ref[0, 0, :][:, None] * ic_ref[0, 0, :][None, :]
    col = j * _CB + lax.broadcasted_iota(jnp.int32, (_RB, _CB), 1)
    s = jnp.where(col < 10000, s, -1e30)
    sim_ref[...] = s
    cm_ref[...] = jnp.max(s.reshape(_RB, 4, 128), axis=2)[None]


def _simk(embp, inv):
    NPAD, D = embp.shape
    gi, gj = NPAD // _RB, NPAD // _CB
    return pl.pallas_call(
        _sim_body,
        grid=(gi, gj),
        in_specs=[
            pl.BlockSpec((_RB, D), lambda i, j: (i, 0)),
            pl.BlockSpec((_CB, D), lambda i, j: (j, 0)),
            pl.BlockSpec((1, 1, _RB), lambda i, j: (i, 0, 0)),
            pl.BlockSpec((1, 1, _CB), lambda i, j: (j, 0, 0)),
        ],
        out_specs=[
            pl.BlockSpec((_RB, _CB), lambda i, j: (i, j)),
            pl.BlockSpec((1, _RB, 4), lambda i, j: (j, i, 0)),
        ],
        out_shape=[
            jax.ShapeDtypeStruct((NPAD, NPAD), jnp.float32),
            jax.ShapeDtypeStruct((gj, NPAD, 4), jnp.float32),
        ],
    )(embp, embp, inv.reshape(gi, 1, _RB), inv.reshape(gj, 1, _CB))


def _tau_body(cm_ref, tau_ref, cmr_ref):
    gj = cm_ref.shape[0]
    v0 = jnp.transpose(cm_ref[...], (1, 0, 2)).reshape(_RB, gj * 4)
    cmr_ref[...] = v0
    v = v0
    for _ in range(15):
        m = jnp.max(v, axis=1)
        v = jnp.where(v == m[:, None], -3.0e38, v)
    tau_ref[...] = jnp.max(v, axis=1)[None, None, :]


def _tauk(cmax):
    gj, NPAD, _ = cmax.shape
    nch = gj * 4
    tau, cmr = pl.pallas_call(
        _tau_body,
        grid=(NPAD // _RB,),
        in_specs=[pl.BlockSpec((gj, _RB, 4), lambda i: (0, i, 0))],
        out_specs=[
            pl.BlockSpec((1, 1, _RB), lambda i: (i, 0, 0)),
            pl.BlockSpec((_RB, nch), lambda i: (i, 0)),
        ],
        out_shape=[
            jax.ShapeDtypeStruct((NPAD // _RB, 1, _RB), jnp.float32),
            jax.ShapeDtypeStruct((NPAD, nch), jnp.float32),
        ],
    )(cmax)
    return tau.reshape(NPAD), cmr


_RWT = _NP // 32
_NCH = _NP // 128
_CCAP = 32


def _topk_body(simv_hbm, cmv_hbm, tau_hbm, out_hbm,
               cm_v, tau_v, clist_v, cbase_v, chunk_v,
               cval_v, cidx_v, obuf_v, sem):
    c = lax.axis_index("c")
    s = lax.axis_index("s")
    t = s * 2 + c
    rbase = t * _RWT
    pltpu.sync_copy(tau_hbm.at[pl.ds(rbase, _RWT)], tau_v)
    iota = lax.iota(jnp.int32, 16)
    zi = jnp.zeros((16,), jnp.int32)
    for u in range(_CCAP // 16):
        clist_v[pl.ds(u * 16, 16)] = zi
    ngrp = jnp.where(rbase + _RWT <= 10000, _RWT // 16,
                     jnp.maximum(0, (10000 - rbase)) // 16)

    def row_fn(r, _):
        row = rbase + r
        pltpu.sync_copy(cmv_hbm.at[pl.ds(row * _NCH, _NCH)], cm_v)
        tsp = plsc.load_gather(tau_v, [jnp.full((16,), r, jnp.int32)])
        cnt = jnp.int32(0)
        for cc in range(_NCH // 16):
            m = cm_v[pl.ds(cc * 16, 16)]
            msk = m >= tsp
            gid = jnp.full((16,), row * _NCH + cc * 16, jnp.int32) + iota
            plsc.store_compressed(clist_v.at[pl.ds(cnt, 16)], gid, mask=msk)
            plsc.store_compressed(cbase_v.at[pl.ds(cnt, 16)],
                                  (iota + cc * 16) * 128, mask=msk)
            pc = jnp.max(plsc.all_reduce_population_count(msk))
            cnt = jnp.minimum(cnt + pc, _CCAP - 16)
        pltpu.async_copy(simv_hbm.at[clist_v], chunk_v, sem).wait()
        neg = jnp.full((16,), -1.0e30)
        for u in range(4):
            cval_v[pl.ds(u * 16, 16)] = neg
        ccnt = jnp.int32(0)

        def scan_fn(q, ccnt):
            cb = plsc.load_gather(cbase_v, [jnp.full((16,), q, jnp.int32)])
            for h in range(8):
                v = chunk_v[q, pl.ds(h * 16, 16)]
                cols = cb + h * 16 + iota
                msk = v >= tsp
                plsc.store_compressed(cval_v.at[pl.ds(ccnt, 16)], v, mask=msk)
                plsc.store_compressed(cidx_v.at[pl.ds(ccnt, 16)], cols, mask=msk)
                pc = jnp.max(plsc.all_reduce_population_count(msk))
                ccnt = jnp.minimum(ccnt + pc, 48)
            return ccnt

        lax.fori_loop(0, cnt, scan_fn, ccnt)
        bv = cval_v[pl.ds(0, 16)]
        bi = cidx_v[pl.ds(0, 16)]
        bv, bi = plsc.sort_key_val(bv, bi)
        for m in range(1, 4):
            nv = cval_v[pl.ds(m * 16, 16)]
            ni = cidx_v[pl.ds(m * 16, 16)]
            nv, ni = plsc.sort_key_val(nv, ni)
            rnv = lax.rev(nv, (0,))
            rni = lax.rev(ni, (0,))
            sel = bv >= rnv
            bv = jnp.where(sel, bv, rnv)
            bi = jnp.where(sel, bi, rni)
            bv, bi = plsc.sort_key_val(bv, bi)
        obuf_v[pl.ds((r % 16) * 16, 16)] = bi

        @pl.when(r % 16 == 15)
        def _flush():
            pltpu.sync_copy(obuf_v, out_hbm.at[pl.ds((row - 15) * 16, 256)])

        return None

    lax.fori_loop(0, ngrp * 16, row_fn, None)


def _topk_sc(simv, cmv, tau):
    mesh = plsc.VectorSubcoreMesh(core_axis_name="c", subcore_axis_name="s")
    f = functools.partial(
        pl.kernel, mesh=mesh,
        out_type=jax.ShapeDtypeStruct((_NP * 16,), jnp.int32),
        compiler_params=pltpu.CompilerParams(needs_layout_passes=False),
        scratch_types=[
            pltpu.VMEM((_NCH,), jnp.float32),
            pltpu.VMEM((_RWT,), jnp.float32),
            pltpu.VMEM((_CCAP,), jnp.int32),
            pltpu.VMEM((_CCAP + 16,), jnp.int32),
            pltpu.VMEM((_CCAP, 128), jnp.float32),
            pltpu.VMEM((64 + 16,), jnp.float32),
            pltpu.VMEM((64 + 16,), jnp.int32),
            pltpu.VMEM((256,), jnp.int32),
            pltpu.SemaphoreType.DMA,
        ],
    )(_topk_body)
    return f(simv, cmv, tau)


def _linscal_body(x_ref, emb_ref, w_ref, att_ref, g_ref, at_ref, bs_ref, ws_ref):
    g = jnp.dot(x_ref[...], w_ref[...], preferred_element_type=jnp.float32)
    g_ref[...] = g
    emb = emb_ref[...]
    att = att_ref[...]
    a_t = g @ att[0, :] + emb @ att[2, :]
    b_s = g @ att[1, :] + emb @ att[3, :]
    at_ref[...] = a_t[None, None, :]
    bs_ref[...] = b_s[None, None, :]
    a = a_t + b_s
    ws_ref[...] = jnp.exp(jnp.where(a >= 0, a, 0.2 * a))[None, None, :]


def _linscal(xb, emb_b, lin_WT, att4):
    BN, L = xb.shape
    D = lin_WT.shape[1]
    blk = 2000
    grid = (BN // blk,)
    g, a_t, b_s, wself = pl.pallas_call(
        _linscal_body,
        grid=grid,
        in_specs=[
            pl.BlockSpec((blk, L), lambda i: (i, 0)),
            pl.BlockSpec((blk, D), lambda i: (i, 0)),
            pl.BlockSpec((L, D), lambda i: (0, 0)),
            pl.BlockSpec((4, D), lambda i: (0, 0)),
        ],
        out_specs=[
            pl.BlockSpec((blk, D), lambda i: (i, 0)),
            pl.BlockSpec((1, 1, blk), lambda i: (i, 0, 0)),
            pl.BlockSpec((1, 1, blk), lambda i: (i, 0, 0)),
            pl.BlockSpec((1, 1, blk), lambda i: (i, 0, 0)),
        ],
        out_shape=[
            jax.ShapeDtypeStruct((BN, D), jnp.float32),
            jax.ShapeDtypeStruct((BN // blk, 1, blk), jnp.float32),
            jax.ShapeDtypeStruct((BN // blk, 1, blk), jnp.float32),
            jax.ShapeDtypeStruct((BN // blk, 1, blk), jnp.float32),
        ],
    )(xb, emb_b, lin_WT, att4)
    return g, a_t.reshape(BN), b_s.reshape(BN), wself.reshape(BN)


def _head_body(num_ref, den_ref, ws_ref, g_ref, emb_ref, gb_ref, bn_ref,
               ow_ref, ob_ref, o_ref):
    ws = ws_ref[0, 0, :]
    den = jnp.sum(den_ref[0], axis=1) + ws + 1e-16
    z = (num_ref[0] + ws[:, None] * g_ref[...]) / den[:, None]
    z = z + gb_ref[...]
    s = z * emb_ref[...]
    s = s * _BNK * bn_ref[0:1, :] + bn_ref[1:2, :]
    s = jnp.maximum(s, 0.0)
    t = jnp.sum(s * ow_ref[...], axis=1)
    o_ref[...] = (t + ob_ref[0, 0])[None, None, :]


def _head(num, den16, wself, g, emb_b, gl_bias, bn2, out_W, out_b):
    D = g.shape[1]
    BN = g.shape[0]
    blk = 2000
    nb = _NP // blk
    out = pl.pallas_call(
        _head_body,
        grid=(BN // blk,),
        in_specs=[
            pl.BlockSpec((1, blk, D), lambda i: (i // nb, i % nb, 0)),
            pl.BlockSpec((1, blk, 16), lambda i: (i // nb, i % nb, 0)),
            pl.BlockSpec((1, 1, blk), lambda i: (i, 0, 0)),
            pl.BlockSpec((blk, D), lambda i: (i, 0)),
            pl.BlockSpec((blk, D), lambda i: (i, 0)),
            pl.BlockSpec((1, D), lambda i: (0, 0)),
            pl.BlockSpec((2, D), lambda i: (0, 0)),
            pl.BlockSpec((1, D), lambda i: (0, 0)),
            pl.BlockSpec((1, 1), lambda i: (0, 0)),
        ],
        out_specs=pl.BlockSpec((1, 1, blk), lambda i: (i, 0, 0)),
        out_shape=jax.ShapeDtypeStruct((BN // blk, 1, blk), jnp.float32),
    )(num, den16, wself.reshape(BN // blk, 1, blk),
      g, emb_b, gl_bias.reshape(1, D), bn2, out_W, out_b.reshape(1, 1))
    return out.reshape(BN)


def kernel(x, emb_table, lin_W, att_i, att_j, att_em_i, att_em_j, gl_bias,
           bn_gamma, bn_beta, out_W, out_b):
    B, N, L = x.shape
    D = emb_table.shape[1]
    BN = B * N

    embp = jnp.zeros((_NP, L), jnp.float32).at[:N].set(emb_table)
    inv = _norms(embp)
    sim, cmax = _simk(embp, inv)
    tau, cmr = _tauk(cmax)
    idxf = _topk_sc(sim.reshape(_NP * _NCH, 128), cmr.reshape(-1), tau)

    xb = x.reshape(BN, L)
    emb_b = jnp.tile(emb_table, (B, 1))
    att4 = jnp.stack([att_i, att_j, att_em_i, att_em_j], axis=0)
    g, a_t, b_s, wself = _linscal(xb, emb_b, lin_W.T, att4)

    a_t2 = jnp.zeros((B, _NP), jnp.float32).at[:, :N].set(a_t.reshape(B, N))
    b_s2 = jnp.zeros((B, _NP), jnp.float32).at[:, :N].set(b_s.reshape(B, N))
    g2 = jnp.zeros((B, _NP, D), jnp.float32).at[:, :N].set(g.reshape(B, N, D))
    zeros_init = jnp.zeros((_RPT, D), jnp.float32)
    num_out, den_out = _edge_sc(idxf, a_t2.reshape(-1),
                                b_s2.reshape(-1), g2.reshape(-1), zeros_init)
    dent = den_out.transpose(0, 2, 1)

    bn2 = jnp.stack([bn_gamma, bn_beta], axis=0)
    out = _head(num_out, dent, wself, g, emb_b, gl_bias, bn2, out_W, out_b)
    return out.reshape(B, N)


# speedup vs baseline: 8.6692x; 3.6798x over previous
import functools

import jax
import jax.numpy as jnp
from jax import lax
from jax.experimental import pallas as pl
from jax.experimental.pallas import tpu as pltpu
from jax.experimental.pallas import tpu_sc as plsc

K = 16
_BNK = float(1.0 / (1.0 + 1e-5) ** 0.5)

_NP = 10240
_RPT = _NP // 16
_GRP = 8
_NGRP = _RPT // _GRP


def _edge_body(idx_hbm, at_hbm, bs_hbm, g_hbm, z_hbm, num_hbm, den_hbm,
               idx_v, at_v, bs_v, gbuf_v, msg_v, tgt_v, den_v, num_sh):
    c = lax.axis_index("c")
    s = lax.axis_index("s")
    base = s * _RPT
    cb = c * _NP

    pltpu.sync_copy(idx_hbm.at[pl.ds(base * K, _RPT * K)], idx_v)
    pltpu.sync_copy(at_hbm.at[pl.ds(cb, _NP)], at_v)
    pltpu.sync_copy(bs_hbm.at[pl.ds(cb + base, _RPT)], bs_v.at[pl.ds(0, _RPT)])
    pltpu.sync_copy(z_hbm, num_sh.at[pl.ds(base, _RPT)])
    zv = jnp.zeros((16,), jnp.float32)

    def zrow(i, _):
        den_v[pl.ds(i * 16, 16)] = zv
        return None

    lax.fori_loop(0, _NP // 16, zrow, None)
    plsc.subcore_barrier()

    def group(gg, _):
        gb = base + gg * _GRP
        pltpu.sync_copy(g_hbm.at[pl.ds((cb + gb) * 128, _GRP * 128)], gbuf_v)
        b16 = bs_v[pl.ds(gg * _GRP, 16)]
        for jj in range(_GRP):
            rl = gg * _GRP + jj
            rowid = base + rl
            okr = jnp.full((16,), rowid < 10000)
            tgt16 = jnp.where(okr, idx_v[pl.ds(rl * K, 16)], 0)
            a16 = plsc.load_gather(at_v, [tgt16])
            al = a16 + jnp.broadcast_to(b16[jj], (16,))
            al = jnp.where(al >= 0, al, 0.2 * al)
            ok = okr & (tgt16 != jnp.full((16,), rowid))
            w = jnp.where(ok, jnp.exp(al), 0.0)
            plsc.addupdate_scatter(den_v, [tgt16], w)
            tgt_v[pl.ds(jj * 16, 16)] = tgt16
            gs = [gbuf_v[pl.ds(jj * 128 + seg * 16, 16)] for seg in range(8)]
            for kk in range(16):
                wk = jnp.broadcast_to(w[kk], (16,))
                row = jj * 16 + kk
                for seg in range(8):
                    msg_v[row, pl.ds(seg * 16, 16)] = gs[seg] * wk
        pltpu.sync_copy(msg_v, num_sh.at[tgt_v], add=True)

    lax.fori_loop(0, _NGRP, group, None)
    pltpu.sync_copy(den_v, den_hbm.at[c, s, :])
    plsc.subcore_barrier()
    pltpu.sync_copy(num_sh.at[pl.ds(base, _RPT)],
                    num_hbm.at[c, pl.ds(base, _RPT), :])


def _edge_sc(idx_flat, a_t2, b_s2, g2flat, zeros_init):
    mesh = plsc.VectorSubcoreMesh(core_axis_name="c", subcore_axis_name="s")
    f = functools.partial(
        pl.kernel, mesh=mesh,
        out_type=[
            jax.ShapeDtypeStruct((2, _NP, 128), jnp.float32),
            jax.ShapeDtypeStruct((2, 16, _NP), jnp.float32),
        ],
        compiler_params=pltpu.CompilerParams(needs_layout_passes=False),
        scratch_types=[
            pltpu.VMEM((_RPT * K,), jnp.int32),
            pltpu.VMEM((_NP,), jnp.float32),
            pltpu.VMEM((_RPT + 16,), jnp.float32),
            pltpu.VMEM((_GRP * 128,), jnp.float32),
            pltpu.VMEM((_GRP * 16, 128), jnp.float32),
            pltpu.VMEM((_GRP * 16,), jnp.int32),
            pltpu.VMEM((_NP,), jnp.float32),
            pltpu.VMEM_SHARED((_NP, 128), jnp.float32),
        ],
    )(_edge_body)
    return f(idx_flat, a_t2, b_s2, g2flat, zeros_init)


def _norm_body(e_ref, o_ref):
    ss = jnp.sum(e_ref[...] * e_ref[...], axis=1)
    o_ref[...] = jnp.where(ss > 0, lax.rsqrt(ss), 0.0)[None, None, :]


def _norms(embp):
    NPAD, D = embp.shape
    blk = 2048
    out = pl.pallas_call(
        _norm_body,
        grid=(NPAD // blk,),
        in_specs=[pl.BlockSpec((blk, D), lambda i: (i, 0))],
        out_specs=pl.BlockSpec((1, 1, blk), lambda i: (i, 0, 0)),
        out_shape=jax.ShapeDtypeStruct((NPAD // blk, 1, blk), jnp.float32),
    )(embp)
    return out.reshape(NPAD)


_RB, _CB = 256, 512


def _sim_body(er_ref, ec_ref, ir_ref, ic_ref, sim_ref, cm_ref):
    j = pl.program_id(1)
    d = lax.dot_general(er_ref[...], ec_ref[...], (((1,), (1,)), ((), ())),
                        preferred_element_type=jnp.float32)
    s = d * ir_ref[0, 0, :][:, None] * ic_ref[0, 0, :][None, :]
    col = j * _CB + lax.broadcasted_iota(jnp.int32, (_RB, _CB), 1)
    s = jnp.where(col < 10000, s, -1e30)
    sim_ref[...] = s
    cm_ref[...] = jnp.max(s.reshape(_RB, 4, 128), axis=2)[None]


def _simk(embp, inv):
    NPAD, D = embp.shape
    gi, gj = NPAD // _RB, NPAD // _CB
    return pl.pallas_call(
        _sim_body,
        grid=(gi, gj),
        in_specs=[
            pl.BlockSpec((_RB, D), lambda i, j: (i, 0)),
            pl.BlockSpec((_CB, D), lambda i, j: (j, 0)),
            pl.BlockSpec((1, 1, _RB), lambda i, j: (i, 0, 0)),
            pl.BlockSpec((1, 1, _CB), lambda i, j: (j, 0, 0)),
        ],
        out_specs=[
            pl.BlockSpec((_RB, _CB), lambda i, j: (i, j)),
            pl.BlockSpec((1, _RB, 4), lambda i, j: (j, i, 0)),
        ],
        out_shape=[
            jax.ShapeDtypeStruct((NPAD, NPAD), jnp.float32),
            jax.ShapeDtypeStruct((gj, NPAD, 4), jnp.float32),
        ],
    )(embp, embp, inv.reshape(gi, 1, _RB), inv.reshape(gj, 1, _CB))


def _tau_body(cm_ref, tau_ref, cmr_ref):
    gj = cm_ref.shape[0]
    v0 = jnp.transpose(cm_ref[...], (1, 0, 2)).reshape(_RB, gj * 4)
    cmr_ref[...] = v0
    v = v0
    for _ in range(15):
        m = jnp.max(v, axis=1)
        v = jnp.where(v == m[:, None], -3.0e38, v)
    tau_ref[...] = jnp.max(v, axis=1)[None, None, :]


def _tauk(cmax):
    gj, NPAD, _ = cmax.shape
    nch = gj * 4
    tau, cmr = pl.pallas_call(
        _tau_body,
        grid=(NPAD // _RB,),
        in_specs=[pl.BlockSpec((gj, _RB, 4), lambda i: (0, i, 0))],
        out_specs=[
            pl.BlockSpec((1, 1, _RB), lambda i: (i, 0, 0)),
            pl.BlockSpec((_RB, nch), lambda i: (i, 0)),
        ],
        out_shape=[
            jax.ShapeDtypeStruct((NPAD // _RB, 1, _RB), jnp.float32),
            jax.ShapeDtypeStruct((NPAD, nch), jnp.float32),
        ],
    )(cmax)
    return tau.reshape(NPAD), cmr


_RWT = _NP // 32
_NCH = _NP // 128
_CCAP = 32


def _topk_body(simv_hbm, cmv_hbm, tau_hbm, out_hbm,
               cm_v, tau_v, cla_v, clb_v, cbase_v, cha_v, chb_v,
               cval_v, cidx_v, obuf_v, sem):
    c = lax.axis_index("c")
    s = lax.axis_index("s")
    t = s * 2 + c
    rbase = t * _RWT
    pltpu.sync_copy(tau_hbm.at[pl.ds(rbase, _RWT)], tau_v)
    iota = lax.iota(jnp.int32, 16)
    ngrp = jnp.where(rbase + _RWT <= 10000, _RWT // 16,
                     jnp.maximum(0, (10000 - rbase)) // 16)

    def grp_fn(q, _):
        rb = q * 16
        row0 = rbase + rb
        pltpu.sync_copy(cmv_hbm.at[pl.ds(row0 * _NCH, 16 * _NCH)], cm_v)

        def make_sel(half, clh_v):
            def sel_fn(r8, _):
                r16 = half * 8 + r8
                row = row0 + r16
                tsp = plsc.load_gather(
                    tau_v, [jnp.full((16,), rb + r16, jnp.int32)])
                clh_v[pl.ds(r8 * 16, 16)] = jnp.full(
                    (16,), row * _NCH + _NCH - 1, jnp.int32)
                cnt = jnp.int32(0)
                for cc in range(_NCH // 16):
                    m = cm_v[pl.ds(r16 * _NCH + cc * 16, 16)]
                    msk = m >= tsp
                    pref = plsc.cumsum(msk.astype(jnp.int32))
                    msk = msk & (pref + cnt <= 16)
                    gid = jnp.full((16,), row * _NCH + cc * 16, jnp.int32) + iota
                    cq = jnp.minimum(cnt, 15)
                    plsc.store_compressed(clh_v.at[pl.ds(r8 * 16 + cq, 16)],
                                          gid, mask=msk)
                    plsc.store_compressed(
                        cbase_v.at[pl.ds(r16 * 16 + cq, 16)],
                        (iota + cc * 16) * 128, mask=msk)
                    pc = jnp.max(plsc.all_reduce_population_count(msk))
                    cnt = cnt + pc
                return None
            return sel_fn

        lax.fori_loop(0, 8, make_sel(0, cla_v), None)
        lax.fori_loop(0, 8, make_sel(1, clb_v), None)
        copy_a = pltpu.async_copy(simv_hbm.at[cla_v], cha_v, sem)
        copy_b = pltpu.async_copy(simv_hbm.at[clb_v], chb_v, sem)
        copy_a.wait()
        copy_b.wait()

        def make_scan(half, chh_v):
            def scan_row(r8, _):
                r16 = half * 8 + r8
                tsp = plsc.load_gather(
                    tau_v, [jnp.full((16,), rb + r16, jnp.int32)])
                neg = jnp.full((16,), -1.0e30)
                for u in range(4):
                    cval_v[pl.ds(u * 16, 16)] = neg

                def chunk_fn(q2, ccnt):
                    cb = plsc.load_gather(
                        cbase_v, [jnp.full((16,), r16 * 16 + q2, jnp.int32)])
                    for h in range(8):
                        v = chh_v[r8 * 16 + q2, pl.ds(h * 16, 16)]
                        cols = cb + h * 16 + iota
                        msk = v >= tsp
                        co = jnp.minimum(ccnt, 48)
                        plsc.store_compressed(cval_v.at[pl.ds(co, 16)], v,
                                              mask=msk)
                        plsc.store_compressed(cidx_v.at[pl.ds(co, 16)], cols,
                                              mask=msk)
                        pc = jnp.max(plsc.all_reduce_population_count(msk))
                        ccnt = jnp.minimum(ccnt + pc, 48)
                    return ccnt

                lax.fori_loop(0, 16, chunk_fn, jnp.int32(0))
                bv = cval_v[pl.ds(0, 16)]
                bi = cidx_v[pl.ds(0, 16)]
                bv, bi = plsc.sort_key_val(bv, bi)
                for m in range(1, 4):
                    nv = cval_v[pl.ds(m * 16, 16)]
                    ni = cidx_v[pl.ds(m * 16, 16)]
                    nv, ni = plsc.sort_key_val(nv, ni)
                    rnv = lax.rev(nv, (0,))
                    rni = lax.rev(ni, (0,))
                    sel = bv >= rnv
                    bv = jnp.where(sel, bv, rnv)
                    bi = jnp.where(sel, bi, rni)
                    bv, bi = plsc.sort_key_val(bv, bi)
                obuf_v[pl.ds(r16 * 16, 16)] = bi
                return None
            return scan_row

        lax.fori_loop(0, 8, make_scan(0, cha_v), None)
        lax.fori_loop(0, 8, make_scan(1, chb_v), None)
        pltpu.sync_copy(obuf_v, out_hbm.at[pl.ds(row0 * 16, 256)])
        return None

    lax.fori_loop(0, ngrp, grp_fn, None)


def _topk_sc(simv, cmv, tau):
    mesh = plsc.VectorSubcoreMesh(core_axis_name="c", subcore_axis_name="s")
    f = functools.partial(
        pl.kernel, mesh=mesh,
        out_type=jax.ShapeDtypeStruct((_NP * 16,), jnp.int32),
        compiler_params=pltpu.CompilerParams(needs_layout_passes=False),
        scratch_types=[
            pltpu.VMEM((16 * _NCH,), jnp.float32),
            pltpu.VMEM((_RWT,), jnp.float32),
            pltpu.VMEM((128,), jnp.int32),
            pltpu.VMEM((128,), jnp.int32),
            pltpu.VMEM((256 + 16,), jnp.int32),
            pltpu.VMEM((128, 128), jnp.float32),
            pltpu.VMEM((128, 128), jnp.float32),
            pltpu.VMEM((64 + 16,), jnp.float32),
            pltpu.VMEM((64 + 16,), jnp.int32),
            pltpu.VMEM((256,), jnp.int32),
            pltpu.SemaphoreType.DMA,
        ],
    )(_topk_body)
    return f(simv, cmv, tau)


def _linscal_body(x_ref, emb_ref, w_ref, att_ref, g_ref, at_ref, bs_ref, ws_ref):
    g = jnp.dot(x_ref[...], w_ref[...], preferred_element_type=jnp.float32)
    g_ref[...] = g
    emb = emb_ref[...]
    att = att_ref[...]
    a_t = g @ att[0, :] + emb @ att[2, :]
    b_s = g @ att[1, :] + emb @ att[3, :]
    at_ref[...] = a_t[None, None, :]
    bs_ref[...] = b_s[None, None, :]
    a = a_t + b_s
    ws_ref[...] = jnp.exp(jnp.where(a >= 0, a, 0.2 * a))[None, None, :]


def _linscal(xb, emb_b, lin_WT, att4):
    BN, L = xb.shape
    D = lin_WT.shape[1]
    blk = 2000
    grid = (BN // blk,)
    g, a_t, b_s, wself = pl.pallas_call(
        _linscal_body,
        grid=grid,
        in_specs=[
            pl.BlockSpec((blk, L), lambda i: (i, 0)),
            pl.BlockSpec((blk, D), lambda i: (i, 0)),
            pl.BlockSpec((L, D), lambda i: (0, 0)),
            pl.BlockSpec((4, D), lambda i: (0, 0)),
        ],
        out_specs=[
            pl.BlockSpec((blk, D), lambda i: (i, 0)),
            pl.BlockSpec((1, 1, blk), lambda i: (i, 0, 0)),
            pl.BlockSpec((1, 1, blk), lambda i: (i, 0, 0)),
            pl.BlockSpec((1, 1, blk), lambda i: (i, 0, 0)),
        ],
        out_shape=[
            jax.ShapeDtypeStruct((BN, D), jnp.float32),
            jax.ShapeDtypeStruct((BN // blk, 1, blk), jnp.float32),
            jax.ShapeDtypeStruct((BN // blk, 1, blk), jnp.float32),
            jax.ShapeDtypeStruct((BN // blk, 1, blk), jnp.float32),
        ],
    )(xb, emb_b, lin_WT, att4)
    return g, a_t.reshape(BN), b_s.reshape(BN), wself.reshape(BN)


def _head_body(num_ref, den_ref, ws_ref, g_ref, emb_ref, gb_ref, bn_ref,
               ow_ref, ob_ref, o_ref):
    ws = ws_ref[0, 0, :]
    den = jnp.sum(den_ref[0], axis=1) + ws + 1e-16
    z = (num_ref[0] + ws[:, None] * g_ref[...]) / den[:, None]
    z = z + gb_ref[...]
    s = z * emb_ref[...]
    s = s * _BNK * bn_ref[0:1, :] + bn_ref[1:2, :]
    s = jnp.maximum(s, 0.0)
    t = jnp.sum(s * ow_ref[...], axis=1)
    o_ref[...] = (t + ob_ref[0, 0])[None, None, :]


def _head(num, den16, wself, g, emb_b, gl_bias, bn2, out_W, out_b):
    D = g.shape[1]
    BN = g.shape[0]
    blk = 2000
    nb = _NP // blk
    out = pl.pallas_call(
        _head_body,
        grid=(BN // blk,),
        in_specs=[
            pl.BlockSpec((1, blk, D), lambda i: (i // nb, i % nb, 0)),
            pl.BlockSpec((1, blk, 16), lambda i: (i // nb, i % nb, 0)),
            pl.BlockSpec((1, 1, blk), lambda i: (i, 0, 0)),
            pl.BlockSpec((blk, D), lambda i: (i, 0)),
            pl.BlockSpec((blk, D), lambda i: (i, 0)),
            pl.BlockSpec((1, D), lambda i: (0, 0)),
            pl.BlockSpec((2, D), lambda i: (0, 0)),
            pl.BlockSpec((1, D), lambda i: (0, 0)),
            pl.BlockSpec((1, 1), lambda i: (0, 0)),
        ],
        out_specs=pl.BlockSpec((1, 1, blk), lambda i: (i, 0, 0)),
        out_shape=jax.ShapeDtypeStruct((BN // blk, 1, blk), jnp.float32),
    )(num, den16, wself.reshape(BN // blk, 1, blk),
      g, emb_b, gl_bias.reshape(1, D), bn2, out_W, out_b.reshape(1, 1))
    return out.reshape(BN)


def kernel(x, emb_table, lin_W, att_i, att_j, att_em_i, att_em_j, gl_bias,
           bn_gamma, bn_beta, out_W, out_b):
    B, N, L = x.shape
    D = emb_table.shape[1]
    BN = B * N

    embp = jnp.zeros((_NP, L), jnp.float32).at[:N].set(emb_table)
    inv = _norms(embp)
    sim, cmax = _simk(embp, inv)
    tau, cmr = _tauk(cmax)
    idxf = _topk_sc(sim.reshape(_NP * _NCH, 128), cmr.reshape(-1), tau)

    xb = x.reshape(BN, L)
    emb_b = jnp.tile(emb_table, (B, 1))
    att4 = jnp.stack([att_i, att_j, att_em_i, att_em_j], axis=0)
    g, a_t, b_s, wself = _linscal(xb, emb_b, lin_W.T, att4)

    a_t2 = jnp.zeros((B, _NP), jnp.float32).at[:, :N].set(a_t.reshape(B, N))
    b_s2 = jnp.zeros((B, _NP), jnp.float32).at[:, :N].set(b_s.reshape(B, N))
    g2 = jnp.zeros((B, _NP, D), jnp.float32).at[:, :N].set(g.reshape(B, N, D))
    zeros_init = jnp.zeros((_RPT, D), jnp.float32)
    num_out, den_out = _edge_sc(idxf, a_t2.reshape(-1),
                                b_s2.reshape(-1), g2.reshape(-1), zeros_init)
    dent = den_out.transpose(0, 2, 1)

    bn2 = jnp.stack([bn_gamma, bn_beta], axis=0)
    out = _head(num_out, dent, wself, g, emb_b, gl_bias, bn2, out_W, out_b)
    return out.reshape(B, N)


# speedup vs baseline: 8.7204x; 1.0059x over previous
import functools

import jax
import jax.numpy as jnp
from jax import lax
from jax.experimental import pallas as pl
from jax.experimental.pallas import tpu as pltpu
from jax.experimental.pallas import tpu_sc as plsc

K = 16
_BNK = float(1.0 / (1.0 + 1e-5) ** 0.5)

_NP = 10240
_RPT = _NP // 16
_GRP = 8
_NGRP = _RPT // _GRP


def _edge_body(idx_hbm, at_hbm, bs_hbm, g_hbm, z_hbm, num_hbm, den_hbm,
               idx_v, at_v, bs_v, gbuf_v, msg_v, tgt_v, den_v, num_sh):
    c = lax.axis_index("c")
    s = lax.axis_index("s")
    base = s * _RPT
    cb = c * _NP

    pltpu.sync_copy(idx_hbm.at[pl.ds(base * K, _RPT * K)], idx_v)
    pltpu.sync_copy(at_hbm.at[pl.ds(cb, _NP)], at_v)
    pltpu.sync_copy(bs_hbm.at[pl.ds(cb + base, _RPT)], bs_v.at[pl.ds(0, _RPT)])
    pltpu.sync_copy(z_hbm, num_sh.at[pl.ds(base, _RPT)])
    zv = jnp.zeros((16,), jnp.float32)

    def zrow(i, _):
        den_v[pl.ds(i * 16, 16)] = zv
        return None

    lax.fori_loop(0, _NP // 16, zrow, None)
    plsc.subcore_barrier()

    def group(gg, _):
        gb = base + gg * _GRP
        pltpu.sync_copy(g_hbm.at[pl.ds((cb + gb) * 128, _GRP * 128)], gbuf_v)
        b16 = bs_v[pl.ds(gg * _GRP, 16)]
        for jj in range(_GRP):
            rl = gg * _GRP + jj
            rowid = base + rl
            okr = jnp.full((16,), rowid < 10000)
            tgt16 = jnp.where(okr, idx_v[pl.ds(rl * K, 16)], 0)
            a16 = plsc.load_gather(at_v, [tgt16])
            al = a16 + jnp.broadcast_to(b16[jj], (16,))
            al = jnp.where(al >= 0, al, 0.2 * al)
            ok = okr & (tgt16 != jnp.full((16,), rowid))
            w = jnp.where(ok, jnp.exp(al), 0.0)
            plsc.addupdate_scatter(den_v, [tgt16], w)
            tgt_v[pl.ds(jj * 16, 16)] = tgt16
            gs = [gbuf_v[pl.ds(jj * 128 + seg * 16, 16)] for seg in range(8)]
            for kk in range(16):
                wk = jnp.broadcast_to(w[kk], (16,))
                row = jj * 16 + kk
                for seg in range(8):
                    msg_v[row, pl.ds(seg * 16, 16)] = gs[seg] * wk
        pltpu.sync_copy(msg_v, num_sh.at[tgt_v], add=True)

    lax.fori_loop(0, _NGRP, group, None)
    pltpu.sync_copy(den_v, den_hbm.at[c, s, :])
    plsc.subcore_barrier()
    pltpu.sync_copy(num_sh.at[pl.ds(base, _RPT)],
                    num_hbm.at[c, pl.ds(base, _RPT), :])


def _edge_sc(idx_flat, a_t2, b_s2, g2flat, zeros_init):
    mesh = plsc.VectorSubcoreMesh(core_axis_name="c", subcore_axis_name="s")
    f = functools.partial(
        pl.kernel, mesh=mesh,
        out_type=[
            jax.ShapeDtypeStruct((2, _NP, 128), jnp.float32),
            jax.ShapeDtypeStruct((2, 16, _NP), jnp.float32),
        ],
        compiler_params=pltpu.CompilerParams(needs_layout_passes=False),
        scratch_types=[
            pltpu.VMEM((_RPT * K,), jnp.int32),
            pltpu.VMEM((_NP,), jnp.float32),
            pltpu.VMEM((_RPT + 16,), jnp.float32),
            pltpu.VMEM((_GRP * 128,), jnp.float32),
            pltpu.VMEM((_GRP * 16, 128), jnp.float32),
            pltpu.VMEM((_GRP * 16,), jnp.int32),
            pltpu.VMEM((_NP,), jnp.float32),
            pltpu.VMEM_SHARED((_NP, 128), jnp.float32),
        ],
    )(_edge_body)
    return f(idx_flat, a_t2, b_s2, g2flat, zeros_init)


def _norm_body(e_ref, o_ref):
    ss = jnp.sum(e_ref[...] * e_ref[...], axis=1)
    o_ref[...] = jnp.where(ss > 0, lax.rsqrt(ss), 0.0)[None, None, :]


def _norms(embp):
    NPAD, D = embp.shape
    blk = 2048
    out = pl.pallas_call(
        _norm_body,
        grid=(NPAD // blk,),
        in_specs=[pl.BlockSpec((blk, D), lambda i: (i, 0))],
        out_specs=pl.BlockSpec((1, 1, blk), lambda i: (i, 0, 0)),
        out_shape=jax.ShapeDtypeStruct((NPAD // blk, 1, blk), jnp.float32),
    )(embp)
    return out.reshape(NPAD)


_RB, _CB = 256, 512


def _sim_body(er_ref, ec_ref, ir_ref, ic_ref, sim_ref, cm_ref):
    j = pl.program_id(1)
    d = lax.dot_general(er_ref[...], ec_ref[...], (((1,), (1,)), ((), ())),
                        preferred_element_type=jnp.float32)
    s = d * ir_ref[0, 0, :][:, None] * ic_ref[0, 0, :][None, :]
    col = j * _CB + lax.broadcasted_iota(jnp.int32, (_RB, _CB), 1)
    s = jnp.where(col < 10000, s, -1e30)
    sim_ref[...] = s
    cm_ref[...] = jnp.max(s.reshape(_RB, 4, 128), axis=2)[None]


def _simk(embp, inv):
    NPAD, D = embp.shape
    gi, gj = NPAD // _RB, NPAD // _CB
    return pl.pallas_call(
        _sim_body,
        grid=(gi, gj),
        in_specs=[
            pl.BlockSpec((_RB, D), lambda i, j: (i, 0)),
            pl.BlockSpec((_CB, D), lambda i, j: (j, 0)),
            pl.BlockSpec((1, 1, _RB), lambda i, j: (i, 0, 0)),
            pl.BlockSpec((1, 1, _CB), lambda i, j: (j, 0, 0)),
        ],
        out_specs=[
            pl.BlockSpec((_RB, _CB), lambda i, j: (i, j)),
            pl.BlockSpec((1, _RB, 4), lambda i, j: (j, i, 0)),
        ],
        out_shape=[
            jax.ShapeDtypeStruct((NPAD, NPAD), jnp.float32),
            jax.ShapeDtypeStruct((gj, NPAD, 4), jnp.float32),
        ],
    )(embp, embp, inv.reshape(gi, 1, _RB), inv.reshape(gj, 1, _CB))


def _tau_body(cm_ref, tau_ref, cmr_ref):
    gj = cm_ref.shape[0]
    v0 = jnp.transpose(cm_ref[...], (1, 0, 2)).reshape(_RB, gj * 4)
    cmr_ref[...] = v0
    v = v0
    for _ in range(15):
        m = jnp.max(v, axis=1)
        v = jnp.where(v == m[:, None], -3.0e38, v)
    tau_ref[...] = jnp.max(v, axis=1)[None, None, :]


def _tauk(cmax):
    gj, NPAD, _ = cmax.shape
    nch = gj * 4
    tau, cmr = pl.pallas_call(
        _tau_body,
        grid=(NPAD // _RB,),
        in_specs=[pl.BlockSpec((gj, _RB, 4), lambda i: (0, i, 0))],
        out_specs=[
            pl.BlockSpec((1, 1, _RB), lambda i: (i, 0, 0)),
            pl.BlockSpec((_RB, nch), lambda i: (i, 0)),
        ],
        out_shape=[
            jax.ShapeDtypeStruct((NPAD // _RB, 1, _RB), jnp.float32),
            jax.ShapeDtypeStruct((NPAD, nch), jnp.float32),
        ],
    )(cmax)
    return tau.reshape(NPAD), cmr


_RWT = _NP // 32
_NCH = _NP // 128
_CCAP = 32


def _topk_body(simv_hbm, cmv_hbm, tau_hbm, out_hbm,
               cm_v, tau_v, cla_v, clb_v, cbase_v, cha_v, chb_v,
               cval_v, cidx_v, obuf_v, sem):
    c = lax.axis_index("c")
    s = lax.axis_index("s")
    t = s * 2 + c
    rbase = t * _RWT
    pltpu.sync_copy(tau_hbm.at[pl.ds(rbase, _RWT)], tau_v)
    iota = lax.iota(jnp.int32, 16)
    ngrp = jnp.where(rbase + _RWT <= 10000, _RWT // 16,
                     jnp.maximum(0, (10000 - rbase)) // 16)

    def grp_fn(q, _):
        rb = q * 16
        row0 = rbase + rb
        pltpu.sync_copy(cmv_hbm.at[pl.ds(row0 * _NCH, 16 * _NCH)], cm_v)

        def make_sel(half, clh_v):
            def sel_fn(r8, _):
                r16 = half * 8 + r8
                row = row0 + r16
                tsp = plsc.load_gather(
                    tau_v, [jnp.full((16,), rb + r16, jnp.int32)])
                clh_v[pl.ds(r8 * 16, 16)] = jnp.full(
                    (16,), row * _NCH + _NCH - 1, jnp.int32)
                cnt = jnp.int32(0)
                for cc in range(_NCH // 16):
                    m = cm_v[pl.ds(r16 * _NCH + cc * 16, 16)]
                    msk = m >= tsp
                    pref = plsc.cumsum(msk.astype(jnp.int32))
                    msk = msk & (pref + cnt <= 16)
                    gid = jnp.full((16,), row * _NCH + cc * 16, jnp.int32) + iota
                    cq = jnp.minimum(cnt, 15)
                    plsc.store_compressed(clh_v.at[pl.ds(r8 * 16 + cq, 16)],
                                          gid, mask=msk)
                    plsc.store_compressed(
                        cbase_v.at[pl.ds(r16 * 16 + cq, 16)],
                        (iota + cc * 16) * 128, mask=msk)
                    pc = jnp.max(plsc.all_reduce_population_count(msk))
                    cnt = cnt + pc
                return None
            return sel_fn

        lax.fori_loop(0, 8, make_sel(0, cla_v), None)
        lax.fori_loop(0, 8, make_sel(1, clb_v), None)
        copy_a = pltpu.async_copy(simv_hbm.at[cla_v], cha_v, sem)
        copy_b = pltpu.async_copy(simv_hbm.at[clb_v], chb_v, sem)

        def make_scan(half, chh_v):
            def scan_row(r8, _):
                r16 = half * 8 + r8
                tsp = plsc.load_gather(
                    tau_v, [jnp.full((16,), rb + r16, jnp.int32)])
                neg = jnp.full((16,), -1.0e30)
                for u in range(4):
                    cval_v[pl.ds(u * 16, 16)] = neg

                def chunk_fn(q2, ccnt):
                    cb = plsc.load_gather(
                        cbase_v, [jnp.full((16,), r16 * 16 + q2, jnp.int32)])
                    for h in range(8):
                        v = chh_v[r8 * 16 + q2, pl.ds(h * 16, 16)]
                        cols = cb + h * 16 + iota
                        msk = v >= tsp
                        co = jnp.minimum(ccnt, 48)
                        plsc.store_compressed(cval_v.at[pl.ds(co, 16)], v,
                                              mask=msk)
                        plsc.store_compressed(cidx_v.at[pl.ds(co, 16)], cols,
                                              mask=msk)
                        pc = jnp.max(plsc.all_reduce_population_count(msk))
                        ccnt = jnp.minimum(ccnt + pc, 48)
                    return ccnt

                lax.fori_loop(0, 16, chunk_fn, jnp.int32(0))
                bv = cval_v[pl.ds(0, 16)]
                bi = cidx_v[pl.ds(0, 16)]
                bv, bi = plsc.sort_key_val(bv, bi)
                for m in range(1, 4):
                    nv = cval_v[pl.ds(m * 16, 16)]
                    ni = cidx_v[pl.ds(m * 16, 16)]
                    nv, ni = plsc.sort_key_val(nv, ni)
                    rnv = lax.rev(nv, (0,))
                    rni = lax.rev(ni, (0,))
                    sel = bv >= rnv
                    bv = jnp.where(sel, bv, rnv)
                    bi = jnp.where(sel, bi, rni)
                    bv, bi = plsc.sort_key_val(bv, bi)
                obuf_v[pl.ds(r16 * 16, 16)] = bi
                return None
            return scan_row

        copy_a.wait()
        lax.fori_loop(0, 8, make_scan(0, cha_v), None)
        copy_b.wait()
        lax.fori_loop(0, 8, make_scan(1, chb_v), None)
        pltpu.sync_copy(obuf_v, out_hbm.at[pl.ds(row0 * 16, 256)])
        return None

    lax.fori_loop(0, ngrp, grp_fn, None)


def _topk_sc(simv, cmv, tau):
    mesh = plsc.VectorSubcoreMesh(core_axis_name="c", subcore_axis_name="s")
    f = functools.partial(
        pl.kernel, mesh=mesh,
        out_type=jax.ShapeDtypeStruct((_NP * 16,), jnp.int32),
        compiler_params=pltpu.CompilerParams(needs_layout_passes=False),
        scratch_types=[
            pltpu.VMEM((16 * _NCH,), jnp.float32),
            pltpu.VMEM((_RWT,), jnp.float32),
            pltpu.VMEM((128,), jnp.int32),
            pltpu.VMEM((128,), jnp.int32),
            pltpu.VMEM((256 + 16,), jnp.int32),
            pltpu.VMEM((128, 128), jnp.float32),
            pltpu.VMEM((128, 128), jnp.float32),
            pltpu.VMEM((64 + 16,), jnp.float32),
            pltpu.VMEM((64 + 16,), jnp.int32),
            pltpu.VMEM((256,), jnp.int32),
            pltpu.SemaphoreType.DMA,
        ],
    )(_topk_body)
    return f(simv, cmv, tau)


def _linscal_body(x_ref, emb_ref, w_ref, att_ref, g_ref, at_ref, bs_ref, ws_ref):
    g = jnp.dot(x_ref[...], w_ref[...], preferred_element_type=jnp.float32)
    g_ref[...] = g
    emb = emb_ref[...]
    att = att_ref[...]
    a_t = g @ att[0, :] + emb @ att[2, :]
    b_s = g @ att[1, :] + emb @ att[3, :]
    at_ref[...] = a_t[None, None, :]
    bs_ref[...] = b_s[None, None, :]
    a = a_t + b_s
    ws_ref[...] = jnp.exp(jnp.where(a >= 0, a, 0.2 * a))[None, None, :]


def _linscal(xb, emb_b, lin_WT, att4):
    BN, L = xb.shape
    D = lin_WT.shape[1]
    blk = 2000
    grid = (BN // blk,)
    g, a_t, b_s, wself = pl.pallas_call(
        _linscal_body,
        grid=grid,
        in_specs=[
            pl.BlockSpec((blk, L), lambda i: (i, 0)),
            pl.BlockSpec((blk, D), lambda i: (i, 0)),
            pl.BlockSpec((L, D), lambda i: (0, 0)),
            pl.BlockSpec((4, D), lambda i: (0, 0)),
        ],
        out_specs=[
            pl.BlockSpec((blk, D), lambda i: (i, 0)),
            pl.BlockSpec((1, 1, blk), lambda i: (i, 0, 0)),
            pl.BlockSpec((1, 1, blk), lambda i: (i, 0, 0)),
            pl.BlockSpec((1, 1, blk), lambda i: (i, 0, 0)),
        ],
        out_shape=[
            jax.ShapeDtypeStruct((BN, D), jnp.float32),
            jax.ShapeDtypeStruct((BN // blk, 1, blk), jnp.float32),
            jax.ShapeDtypeStruct((BN // blk, 1, blk), jnp.float32),
            jax.ShapeDtypeStruct((BN // blk, 1, blk), jnp.float32),
        ],
    )(xb, emb_b, lin_WT, att4)
    return g, a_t.reshape(BN), b_s.reshape(BN), wself.reshape(BN)


def _head_body(num_ref, den_ref, ws_ref, g_ref, emb_ref, gb_ref, bn_ref,
               ow_ref, ob_ref, o_ref):
    ws = ws_ref[0, 0, :]
    den = jnp.sum(den_ref[0], axis=1) + ws + 1e-16
    z = (num_ref[0] + ws[:, None] * g_ref[...]) / den[:, None]
    z = z + gb_ref[...]
    s = z * emb_ref[...]
    s = s * _BNK * bn_ref[0:1, :] + bn_ref[1:2, :]
    s = jnp.maximum(s, 0.0)
    t = jnp.sum(s * ow_ref[...], axis=1)
    o_ref[...] = (t + ob_ref[0, 0])[None, None, :]


def _head(num, den16, wself, g, emb_b, gl_bias, bn2, out_W, out_b):
    D = g.shape[1]
    BN = g.shape[0]
    blk = 2000
    nb = _NP // blk
    out = pl.pallas_call(
        _head_body,
        grid=(BN // blk,),
        in_specs=[
            pl.BlockSpec((1, blk, D), lambda i: (i // nb, i % nb, 0)),
            pl.BlockSpec((1, blk, 16), lambda i: (i // nb, i % nb, 0)),
            pl.BlockSpec((1, 1, blk), lambda i: (i, 0, 0)),
            pl.BlockSpec((blk, D), lambda i: (i, 0)),
            pl.BlockSpec((blk, D), lambda i: (i, 0)),
            pl.BlockSpec((1, D), lambda i: (0, 0)),
            pl.BlockSpec((2, D), lambda i: (0, 0)),
            pl.BlockSpec((1, D), lambda i: (0, 0)),
            pl.BlockSpec((1, 1), lambda i: (0, 0)),
        ],
        out_specs=pl.BlockSpec((1, 1, blk), lambda i: (i, 0, 0)),
        out_shape=jax.ShapeDtypeStruct((BN // blk, 1, blk), jnp.float32),
    )(num, den16, wself.reshape(BN // blk, 1, blk),
      g, emb_b, gl_bias.reshape(1, D), bn2, out_W, out_b.reshape(1, 1))
    return out.reshape(BN)


def kernel(x, emb_table, lin_W, att_i, att_j, att_em_i, att_em_j, gl_bias,
           bn_gamma, bn_beta, out_W, out_b):
    B, N, L = x.shape
    D = emb_table.shape[1]
    BN = B * N

    embp = jnp.zeros((_NP, L), jnp.float32).at[:N].set(emb_table)
    inv = _norms(embp)
    sim, cmax = _simk(embp, inv)
    tau, cmr = _tauk(cmax)
    idxf = _topk_sc(sim.reshape(_NP * _NCH, 128), cmr.reshape(-1), tau)

    xb = x.reshape(BN, L)
    emb_b = jnp.tile(emb_table, (B, 1))
    att4 = jnp.stack([att_i, att_j, att_em_i, att_em_j], axis=0)
    g, a_t, b_s, wself = _linscal(xb, emb_b, lin_W.T, att4)

    a_t2 = jnp.zeros((B, _NP), jnp.float32).at[:, :N].set(a_t.reshape(B, N))
    b_s2 = jnp.zeros((B, _NP), jnp.float32).at[:, :N].set(b_s.reshape(B, N))
    g2 = jnp.zeros((B, _NP, D), jnp.float32).at[:, :N].set(g.reshape(B, N, D))
    zeros_init = jnp.zeros((_RPT, D), jnp.float32)
    num_out, den_out = _edge_sc(idxf, a_t2.reshape(-1),
                                b_s2.reshape(-1), g2.reshape(-1), zeros_init)
    dent = den_out.transpose(0, 2, 1)

    bn2 = jnp.stack([bn_gamma, bn_beta], axis=0)
    out = _head(num_out, dent, wself, g, emb_b, gl_bias, bn2, out_W, out_b)
    return out.reshape(B, N)


# speedup vs baseline: 8.7741x; 1.0062x over previous
import functools

import jax
import jax.numpy as jnp
from jax import lax
from jax.experimental import pallas as pl
from jax.experimental.pallas import tpu as pltpu
from jax.experimental.pallas import tpu_sc as plsc

K = 16
_BNK = float(1.0 / (1.0 + 1e-5) ** 0.5)

_NP = 10240
_RPT = _NP // 16
_GRP = 8
_NGRP = _RPT // _GRP


def _edge_body(idx_hbm, at_hbm, bs_hbm, g_hbm, z_hbm, num_hbm, den_hbm,
               idx_v, at_v, bs_v, gbuf_v, msg_v, tgt_v, den_v, num_sh):
    c = lax.axis_index("c")
    s = lax.axis_index("s")
    base = s * _RPT
    cb = c * _NP

    pltpu.sync_copy(idx_hbm.at[pl.ds(base * K, _RPT * K)], idx_v)
    pltpu.sync_copy(at_hbm.at[pl.ds(cb, _NP)], at_v)
    pltpu.sync_copy(bs_hbm.at[pl.ds(cb + base, _RPT)], bs_v.at[pl.ds(0, _RPT)])
    pltpu.sync_copy(z_hbm, num_sh.at[pl.ds(base, _RPT)])
    zv = jnp.zeros((16,), jnp.float32)

    def zrow(i, _):
        den_v[pl.ds(i * 16, 16)] = zv
        return None

    lax.fori_loop(0, _NP // 16, zrow, None)
    plsc.subcore_barrier()

    def group(gg, _):
        gb = base + gg * _GRP
        pltpu.sync_copy(g_hbm.at[pl.ds((cb + gb) * 128, _GRP * 128)], gbuf_v)
        b16 = bs_v[pl.ds(gg * _GRP, 16)]
        for jj in range(_GRP):
            rl = gg * _GRP + jj
            rowid = base + rl
            okr = jnp.full((16,), rowid < 10000)
            tgt16 = jnp.where(okr, idx_v[pl.ds(rl * K, 16)], 0)
            a16 = plsc.load_gather(at_v, [tgt16])
            al = a16 + jnp.broadcast_to(b16[jj], (16,))
            al = jnp.where(al >= 0, al, 0.2 * al)
            ok = okr & (tgt16 != jnp.full((16,), rowid))
            w = jnp.where(ok, jnp.exp(al), 0.0)
            plsc.addupdate_scatter(den_v, [tgt16], w)
            tgt_v[pl.ds(jj * 16, 16)] = tgt16
            gs = [gbuf_v[pl.ds(jj * 128 + seg * 16, 16)] for seg in range(8)]
            for kk in range(16):
                wk = jnp.broadcast_to(w[kk], (16,))
                row = jj * 16 + kk
                for seg in range(8):
                    msg_v[row, pl.ds(seg * 16, 16)] = gs[seg] * wk
        pltpu.sync_copy(msg_v, num_sh.at[tgt_v], add=True)

    lax.fori_loop(0, _NGRP, group, None)
    pltpu.sync_copy(den_v, den_hbm.at[c, s, :])
    plsc.subcore_barrier()
    pltpu.sync_copy(num_sh.at[pl.ds(base, _RPT)],
                    num_hbm.at[c, pl.ds(base, _RPT), :])


def _edge_sc(idx_flat, a_t2, b_s2, g2flat, zeros_init):
    mesh = plsc.VectorSubcoreMesh(core_axis_name="c", subcore_axis_name="s")
    f = functools.partial(
        pl.kernel, mesh=mesh,
        out_type=[
            jax.ShapeDtypeStruct((2, _NP, 128), jnp.float32),
            jax.ShapeDtypeStruct((2, 16, _NP), jnp.float32),
        ],
        compiler_params=pltpu.CompilerParams(needs_layout_passes=False),
        scratch_types=[
            pltpu.VMEM((_RPT * K,), jnp.int32),
            pltpu.VMEM((_NP,), jnp.float32),
            pltpu.VMEM((_RPT + 16,), jnp.float32),
            pltpu.VMEM((_GRP * 128,), jnp.float32),
            pltpu.VMEM((_GRP * 16, 128), jnp.float32),
            pltpu.VMEM((_GRP * 16,), jnp.int32),
            pltpu.VMEM((_NP,), jnp.float32),
            pltpu.VMEM_SHARED((_NP, 128), jnp.float32),
        ],
    )(_edge_body)
    return f(idx_flat, a_t2, b_s2, g2flat, zeros_init)


def _norm_body(e_ref, o_ref):
    ss = jnp.sum(e_ref[...] * e_ref[...], axis=1)
    o_ref[...] = jnp.sqrt(ss)[None, None, :]


def _norms(embp):
    NPAD, D = embp.shape
    blk = 2048
    out = pl.pallas_call(
        _norm_body,
        grid=(NPAD // blk,),
        in_specs=[pl.BlockSpec((blk, D), lambda i: (i, 0))],
        out_specs=pl.BlockSpec((1, 1, blk), lambda i: (i, 0, 0)),
        out_shape=jax.ShapeDtypeStruct((NPAD // blk, 1, blk), jnp.float32),
    )(embp)
    return out.reshape(NPAD)


_RB, _CB = 256, 512


def _sim_body(er_ref, ec_ref, ir_ref, ic_ref, sim_ref, cm_ref):
    j = pl.program_id(1)
    d = lax.dot_general(er_ref[...], ec_ref[...], (((1,), (1,)), ((), ())),
                        preferred_element_type=jnp.float32)
    s = d / (ir_ref[0, 0, :][:, None] * ic_ref[0, 0, :][None, :])
    col = j * _CB + lax.broadcasted_iota(jnp.int32, (_RB, _CB), 1)
    s = jnp.where(col < 10000, s, -1e30)
    sim_ref[...] = s
    cm_ref[...] = jnp.max(s.reshape(_RB, 4, 128), axis=2)[None]


def _simk(embp, inv):
    NPAD, D = embp.shape
    gi, gj = NPAD // _RB, NPAD // _CB
    return pl.pallas_call(
        _sim_body,
        grid=(gi, gj),
        in_specs=[
            pl.BlockSpec((_RB, D), lambda i, j: (i, 0)),
            pl.BlockSpec((_CB, D), lambda i, j: (j, 0)),
            pl.BlockSpec((1, 1, _RB), lambda i, j: (i, 0, 0)),
            pl.BlockSpec((1, 1, _CB), lambda i, j: (j, 0, 0)),
        ],
        out_specs=[
            pl.BlockSpec((_RB, _CB), lambda i, j: (i, j)),
            pl.BlockSpec((1, _RB, 4), lambda i, j: (j, i, 0)),
        ],
        out_shape=[
            jax.ShapeDtypeStruct((NPAD, NPAD), jnp.float32),
            jax.ShapeDtypeStruct((gj, NPAD, 4), jnp.float32),
        ],
    )(embp, embp, inv.reshape(gi, 1, _RB), inv.reshape(gj, 1, _CB))


def _tau_body(cm_ref, tau_ref, cmr_ref):
    gj = cm_ref.shape[0]
    v0 = jnp.transpose(cm_ref[...], (1, 0, 2)).reshape(_RB, gj * 4)
    cmr_ref[...] = v0
    v = v0
    for _ in range(15):
        m = jnp.max(v, axis=1)
        v = jnp.where(v == m[:, None], -3.0e38, v)
    tau_ref[...] = jnp.max(v, axis=1)[None, None, :]


def _tauk(cmax):
    gj, NPAD, _ = cmax.shape
    nch = gj * 4
    tau, cmr = pl.pallas_call(
        _tau_body,
        grid=(NPAD // _RB,),
        in_specs=[pl.BlockSpec((gj, _RB, 4), lambda i: (0, i, 0))],
        out_specs=[
            pl.BlockSpec((1, 1, _RB), lambda i: (i, 0, 0)),
            pl.BlockSpec((_RB, nch), lambda i: (i, 0)),
        ],
        out_shape=[
            jax.ShapeDtypeStruct((NPAD // _RB, 1, _RB), jnp.float32),
            jax.ShapeDtypeStruct((NPAD, nch), jnp.float32),
        ],
    )(cmax)
    return tau.reshape(NPAD), cmr


_RWT = _NP // 32
_NCH = _NP // 128
_CCAP = 32


def _topk_body(simv_hbm, cmv_hbm, tau_hbm, out_hbm,
               cm_v, tau_v, cla_v, clb_v, cbase_v, cha_v, chb_v,
               cval_v, cidx_v, obuf_v, sem):
    c = lax.axis_index("c")
    s = lax.axis_index("s")
    t = s * 2 + c
    rbase = t * _RWT
    pltpu.sync_copy(tau_hbm.at[pl.ds(rbase, _RWT)], tau_v)
    iota = lax.iota(jnp.int32, 16)
    ngrp = jnp.where(rbase + _RWT <= 10000, _RWT // 16,
                     jnp.maximum(0, (10000 - rbase)) // 16)

    def grp_fn(q, _):
        rb = q * 16
        row0 = rbase + rb
        pltpu.sync_copy(cmv_hbm.at[pl.ds(row0 * _NCH, 16 * _NCH)], cm_v)

        def make_sel(half, clh_v):
            def sel_fn(r8, _):
                r16 = half * 8 + r8
                row = row0 + r16
                tsp = plsc.load_gather(
                    tau_v, [jnp.full((16,), rb + r16, jnp.int32)])
                clh_v[pl.ds(r8 * 16, 16)] = jnp.full(
                    (16,), row * _NCH + _NCH - 1, jnp.int32)
                cnt = jnp.int32(0)
                for cc in range(_NCH // 16):
                    m = cm_v[pl.ds(r16 * _NCH + cc * 16, 16)]
                    msk = m >= tsp
                    pref = plsc.cumsum(msk.astype(jnp.int32))
                    msk = msk & (pref + cnt <= 16)
                    gid = jnp.full((16,), row * _NCH + cc * 16, jnp.int32) + iota
                    cq = jnp.minimum(cnt, 15)
                    plsc.store_compressed(clh_v.at[pl.ds(r8 * 16 + cq, 16)],
                                          gid, mask=msk)
                    plsc.store_compressed(
                        cbase_v.at[pl.ds(r16 * 16 + cq, 16)],
                        (iota + cc * 16) * 128, mask=msk)
                    pc = jnp.max(plsc.all_reduce_population_count(msk))
                    cnt = cnt + pc
                return None
            return sel_fn

        lax.fori_loop(0, 8, make_sel(0, cla_v), None)
        lax.fori_loop(0, 8, make_sel(1, clb_v), None)
        copy_a = pltpu.async_copy(simv_hbm.at[cla_v], cha_v, sem)
        copy_b = pltpu.async_copy(simv_hbm.at[clb_v], chb_v, sem)

        def make_scan(half, chh_v):
            def scan_row(r8, _):
                r16 = half * 8 + r8
                tsp = plsc.load_gather(
                    tau_v, [jnp.full((16,), rb + r16, jnp.int32)])
                neg = jnp.full((16,), -1.0e30)
                for u in range(4):
                    cval_v[pl.ds(u * 16, 16)] = neg

                def chunk_fn(q2, ccnt):
                    cb = plsc.load_gather(
                        cbase_v, [jnp.full((16,), r16 * 16 + q2, jnp.int32)])
                    for h in range(8):
                        v = chh_v[r8 * 16 + q2, pl.ds(h * 16, 16)]
                        cols = cb + h * 16 + iota
                        msk = v >= tsp
                        co = jnp.minimum(ccnt, 48)
                        plsc.store_compressed(cval_v.at[pl.ds(co, 16)], v,
                                              mask=msk)
                        plsc.store_compressed(cidx_v.at[pl.ds(co, 16)], cols,
                                              mask=msk)
                        pc = jnp.max(plsc.all_reduce_population_count(msk))
                        ccnt = jnp.minimum(ccnt + pc, 48)
                    return ccnt

                lax.fori_loop(0, 16, chunk_fn, jnp.int32(0))
                bv = cval_v[pl.ds(0, 16)]
                bi = cidx_v[pl.ds(0, 16)]
                bv, bi = plsc.sort_key_val(bv, bi)
                for m in range(1, 4):
                    nv = cval_v[pl.ds(m * 16, 16)]
                    ni = cidx_v[pl.ds(m * 16, 16)]
                    nv, ni = plsc.sort_key_val(nv, ni)
                    rnv = lax.rev(nv, (0,))
                    rni = lax.rev(ni, (0,))
                    sel = bv >= rnv
                    bv = jnp.where(sel, bv, rnv)
                    bi = jnp.where(sel, bi, rni)
                    bv, bi = plsc.sort_key_val(bv, bi)
                obuf_v[pl.ds(r16 * 16, 16)] = bi
                return None
            return scan_row

        copy_a.wait()
        lax.fori_loop(0, 8, make_scan(0, cha_v), None)
        copy_b.wait()
        lax.fori_loop(0, 8, make_scan(1, chb_v), None)
        pltpu.sync_copy(obuf_v, out_hbm.at[pl.ds(row0 * 16, 256)])
        return None

    lax.fori_loop(0, ngrp, grp_fn, None)


def _topk_sc(simv, cmv, tau):
    mesh = plsc.VectorSubcoreMesh(core_axis_name="c", subcore_axis_name="s")
    f = functools.partial(
        pl.kernel, mesh=mesh,
        out_type=jax.ShapeDtypeStruct((_NP * 16,), jnp.int32),
        compiler_params=pltpu.CompilerParams(needs_layout_passes=False),
        scratch_types=[
            pltpu.VMEM((16 * _NCH,), jnp.float32),
            pltpu.VMEM((_RWT,), jnp.float32),
            pltpu.VMEM((128,), jnp.int32),
            pltpu.VMEM((128,), jnp.int32),
            pltpu.VMEM((256 + 16,), jnp.int32),
            pltpu.VMEM((128, 128), jnp.float32),
            pltpu.VMEM((128, 128), jnp.float32),
            pltpu.VMEM((64 + 16,), jnp.float32),
            pltpu.VMEM((64 + 16,), jnp.int32),
            pltpu.VMEM((256,), jnp.int32),
            pltpu.SemaphoreType.DMA,
        ],
    )(_topk_body)
    return f(simv, cmv, tau)


def _linscal_body(x_ref, emb_ref, w_ref, att_ref, g_ref, at_ref, bs_ref, ws_ref):
    g = jnp.dot(x_ref[...], w_ref[...], preferred_element_type=jnp.float32)
    g_ref[...] = g
    emb = emb_ref[...]
    att = att_ref[...]
    a_t = g @ att[0, :] + emb @ att[2, :]
    b_s = g @ att[1, :] + emb @ att[3, :]
    at_ref[...] = a_t[None, None, :]
    bs_ref[...] = b_s[None, None, :]
    a = a_t + b_s
    ws_ref[...] = jnp.exp(jnp.where(a >= 0, a, 0.2 * a))[None, None, :]


def _linscal(xb, emb_b, lin_WT, att4):
    BN, L = xb.shape
    D = lin_WT.shape[1]
    blk = 2000
    grid = (BN // blk,)
    g, a_t, b_s, wself = pl.pallas_call(
        _linscal_body,
        grid=grid,
        in_specs=[
            pl.BlockSpec((blk, L), lambda i: (i, 0)),
            pl.BlockSpec((blk, D), lambda i: (i, 0)),
            pl.BlockSpec((L, D), lambda i: (0, 0)),
            pl.BlockSpec((4, D), lambda i: (0, 0)),
        ],
        out_specs=[
            pl.BlockSpec((blk, D), lambda i: (i, 0)),
            pl.BlockSpec((1, 1, blk), lambda i: (i, 0, 0)),
            pl.BlockSpec((1, 1, blk), lambda i: (i, 0, 0)),
            pl.BlockSpec((1, 1, blk), lambda i: (i, 0, 0)),
        ],
        out_shape=[
            jax.ShapeDtypeStruct((BN, D), jnp.float32),
            jax.ShapeDtypeStruct((BN // blk, 1, blk), jnp.float32),
            jax.ShapeDtypeStruct((BN // blk, 1, blk), jnp.float32),
            jax.ShapeDtypeStruct((BN // blk, 1, blk), jnp.float32),
        ],
    )(xb, emb_b, lin_WT, att4)
    return g, a_t.reshape(BN), b_s.reshape(BN), wself.reshape(BN)


def _head_body(num_ref, den_ref, ws_ref, g_ref, emb_ref, gb_ref, bn_ref,
               ow_ref, ob_ref, o_ref):
    ws = ws_ref[0, 0, :]
    den = jnp.sum(den_ref[0], axis=1) + ws + 1e-16
    z = (num_ref[0] + ws[:, None] * g_ref[...]) / den[:, None]
    z = z + gb_ref[...]
    s = z * emb_ref[...]
    s = s * _BNK * bn_ref[0:1, :] + bn_ref[1:2, :]
    s = jnp.maximum(s, 0.0)
    t = jnp.sum(s * ow_ref[...], axis=1)
    o_ref[...] = (t + ob_ref[0, 0])[None, None, :]


def _head(num, den16, wself, g, emb_b, gl_bias, bn2, out_W, out_b):
    D = g.shape[1]
    BN = g.shape[0]
    blk = 2000
    nb = _NP // blk
    out = pl.pallas_call(
        _head_body,
        grid=(BN // blk,),
        in_specs=[
            pl.BlockSpec((1, blk, D), lambda i: (i // nb, i % nb, 0)),
            pl.BlockSpec((1, blk, 16), lambda i: (i // nb, i % nb, 0)),
            pl.BlockSpec((1, 1, blk), lambda i: (i, 0, 0)),
            pl.BlockSpec((blk, D), lambda i: (i, 0)),
            pl.BlockSpec((blk, D), lambda i: (i, 0)),
            pl.BlockSpec((1, D), lambda i: (0, 0)),
            pl.BlockSpec((2, D), lambda i: (0, 0)),
            pl.BlockSpec((1, D), lambda i: (0, 0)),
            pl.BlockSpec((1, 1), lambda i: (0, 0)),
        ],
        out_specs=pl.BlockSpec((1, 1, blk), lambda i: (i, 0, 0)),
        out_shape=jax.ShapeDtypeStruct((BN // blk, 1, blk), jnp.float32),
    )(num, den16, wself.reshape(BN // blk, 1, blk),
      g, emb_b, gl_bias.reshape(1, D), bn2, out_W, out_b.reshape(1, 1))
    return out.reshape(BN)


def kernel(x, emb_table, lin_W, att_i, att_j, att_em_i, att_em_j, gl_bias,
           bn_gamma, bn_beta, out_W, out_b):
    B, N, L = x.shape
    D = emb_table.shape[1]
    BN = B * N

    embp = jnp.zeros((_NP, L), jnp.float32).at[:N].set(emb_table)
    inv = _norms(embp)
    sim, cmax = _simk(embp, inv)
    tau, cmr = _tauk(cmax)
    idxf = _topk_sc(sim.reshape(_NP * _NCH, 128), cmr.reshape(-1), tau)

    xb = x.reshape(BN, L)
    emb_b = jnp.tile(emb_table, (B, 1))
    att4 = jnp.stack([att_i, att_j, att_em_i, att_em_j], axis=0)
    g, a_t, b_s, wself = _linscal(xb, emb_b, lin_W.T, att4)

    a_t2 = jnp.zeros((B, _NP), jnp.float32).at[:, :N].set(a_t.reshape(B, N))
    b_s2 = jnp.zeros((B, _NP), jnp.float32).at[:, :N].set(b_s.reshape(B, N))
    g2 = jnp.zeros((B, _NP, D), jnp.float32).at[:, :N].set(g.reshape(B, N, D))
    zeros_init = jnp.zeros((_RPT, D), jnp.float32)
    num_out, den_out = _edge_sc(idxf, a_t2.reshape(-1),
                                b_s2.reshape(-1), g2.reshape(-1), zeros_init)
    dent = den_out.transpose(0, 2, 1)

    bn2 = jnp.stack([bn_gamma, bn_beta], axis=0)
    out = _head(num_out, dent, wself, g, emb_b, gl_bias, bn2, out_W, out_b)
    return out.reshape(B, N)
